# baseline (speedup 1.0000x reference)
import functools

import jax
import jax.numpy as jnp
from jax import lax
from jax.experimental import pallas as pl
from jax.experimental.pallas import tpu as pltpu
from jax.experimental.pallas import tpu_sc as plsc

_NC, _NS, _L = 2, 16, 16
_NW = _NC * _NS
_B = 128
_LODBITS = (4, 5, 6, 7)
_PAIRS = [(dx, dy) for dx in (0, 1) for dy in (0, 1)]


@functools.lru_cache(maxsize=None)
def _make_kernel(n: int):
    tb = -(-n // _B)
    kmax = (-(-tb // _NW) + 1) // 2

    mesh = plsc.VectorSubcoreMesh(core_axis_name="c", subcore_axis_name="s",
                                  num_cores=_NC, num_subcores=_NS)

    def body(pts, t0, t1, t2, t3, out,
             ptsb0, ptsb1, fx0, fy0, fz0, fx1, fy1, fz1, idx0, idx1,
             rows0, rows1, outv0, outv1,
             g00, g01, g02, g03, g10, g11, g12, g13,
             xs0, xs1, os0, os1):
        tabs = [t0, t1, t2, t3]
        ptsb = [ptsb0, ptsb1]
        fxs = [fx0, fx1]
        fys = [fy0, fy1]
        fzs = [fz0, fz1]
        idxs = [idx0, idx1]
        rows = [rows0, rows1]
        outv = [outv0, outv1]
        gsem = [[g00, g01, g02, g03], [g10, g11, g12, g13]]
        xsem = [xs0, xs1]
        osem = [os0, os1]

        wid = lax.axis_index("s") * _NC + lax.axis_index("c")
        lo = wid * tb // _NW
        hi = (wid + 1) * tb // _NW
        nw = hi - lo

        iota = lax.iota(jnp.int32, _L)
        feat8 = iota & 7
        feat8h = feat8 + 8
        rep8 = iota >> 3
        def r0_of(m):
            return jnp.minimum((lo + m) * _B, n - _B)

        def fire_xyz(m, h):
            pltpu.async_copy(pts.at[pl.ds(r0_of(m) * 3, _B * 3)],
                             ptsb[h], xsem[h])

        def wait_xyz(h):
            pltpu.make_async_copy(
                pts.at[pl.ds(0, _B * 3)], ptsb[h], xsem[h]).wait()

        def stage(m, h):
            for l in range(4):
                lb = _LODBITS[l]
                lod = 1 << lb
                scale = (lod - 1) * 0.5

                def idx_body(i, c, l=l, lb=lb, lod=lod, scale=scale):
                    row3 = (i * _L + iota) * 3
                    x = plsc.load_gather(ptsb[h], [row3])
                    y = plsc.load_gather(ptsb[h], [row3 + 1])
                    z = plsc.load_gather(ptsb[h], [row3 + 2])
                    x = x * scale + scale
                    y = y * scale + scale
                    z = z * scale + scale
                    xi = jnp.minimum(x.astype(jnp.int32), lod - 2)
                    yi = jnp.minimum(y.astype(jnp.int32), lod - 2)
                    zi = jnp.minimum(z.astype(jnp.int32), lod - 2)
                    fsl = pl.ds(l * _B + i * _L, _L)
                    fxs[h][fsl] = x - xi.astype(jnp.float32)
                    fys[h][fsl] = y - yi.astype(jnp.float32)
                    fzs[h][fsl] = z - zi.astype(jnp.float32)
                    bidx = (xi << (2 * lb)) + (yi << lb) + zi
                    for c2, (dx, dy) in enumerate(_PAIRS):
                        off = dx * lod * lod + dy * lod
                        isl = pl.ds((l * 4 + c2) * _B + i * _L, _L)
                        idxs[h][isl] = bidx + off
                    return c

                lax.fori_loop(0, _B // _L, idx_body, 0)
                for c2 in range(4):
                    s = pl.ds((l * 4 + c2) * _B, _B)
                    pltpu.async_copy(tabs[l].at[idxs[h].at[s]],
                                     rows[h].at[s], gsem[h][l])

        def drain_gathers(h, l):
            for _ in range(4):
                pltpu.make_async_copy(
                    tabs[l].at[pl.ds(0, _B)],
                    rows[h].at[pl.ds(0, _B)], gsem[h][l]).wait()

        def drain_out(h):
            pltpu.make_async_copy(
                outv[h], out.at[pl.ds(0, _B * 32)], osem[h]).wait()

        def accum(m, h):
            for l in range(4):
                drain_gathers(h, l)
                opatt = feat8 + (rep8 << 5) + l * 8

                def acc_body(q, c, l=l, opatt=opatt):
                    pvec = q * 2 + rep8
                    fvec = l * _B + pvec
                    fxe = plsc.load_gather(fxs[h], [fvec])
                    fye = plsc.load_gather(fys[h], [fvec])
                    fze = plsc.load_gather(fzs[h], [fvec])
                    gxe = 1.0 - fxe
                    gye = 1.0 - fye
                    gze = 1.0 - fze
                    wxy = ((gxe * gye, gxe * fye), (fxe * gye, fxe * fye))
                    acc = None
                    for c2, (dx, dy) in enumerate(_PAIRS):
                        prow = (l * 4 + c2) * _B + pvec
                        rz0 = plsc.load_gather(rows[h], [prow, feat8])
                        rz1 = plsc.load_gather(rows[h], [prow, feat8h])
                        t = gze * rz0 + fze * rz1
                        w = wxy[dx][dy]
                        acc = w * t if acc is None else acc + w * t
                    plsc.store_scatter(outv[h], [q * 64 + opatt], acc)
                    return c

                lax.fori_loop(0, _B // 2, acc_body, 0)
            pltpu.async_copy(outv[h], out.at[pl.ds(r0_of(m) * 32, _B * 32)],
                             osem[h])

        def loop(k, carry):
            @pl.when(k < nw)
            def _():
                fire_xyz(k, 0)
                wait_xyz(0)
                stage(k, 0)
                accum(k, 0)
                drain_out(0)
            return carry

        lax.fori_loop(0, 2 * kmax, loop, 0)

    return pl.kernel(
        body,
        out_type=jax.ShapeDtypeStruct((n * 32,), jnp.float32),
        mesh=mesh,
        compiler_params=pltpu.CompilerParams(
            needs_layout_passes=False, use_tc_tiling_on_sc=False),
        scratch_types=(
            [pltpu.VMEM((_B * 3,), jnp.float32)] * 2
            + [pltpu.VMEM((4 * _B,), jnp.float32)] * 6
            + [pltpu.VMEM((4 * 4 * _B,), jnp.int32)] * 2
            + [pltpu.VMEM((4 * 4 * _B, 2 * 8), jnp.float32)] * 2
            + [pltpu.VMEM((_B * 32,), jnp.float32)] * 2
            + [pltpu.SemaphoreType.DMA] * 12
        ),
    )


def _zpair(g):
    return jnp.concatenate([g, jnp.roll(g, -1, axis=0)], axis=1)


def kernel(pts, grid_0, grid_1, grid_2, grid_3):
    n = pts.shape[0]
    tabs = [_zpair(g) for g in (grid_0, grid_1, grid_2, grid_3)]
    out = _make_kernel(n)(pts.reshape(n * 3), *tabs)
    return out.reshape(n, 32)
